# baseline (speedup 1.0000x reference)
import functools
import math

import jax
import jax.numpy as jnp
from jax import lax
from jax.experimental import pallas as pl
from jax.experimental.pallas import tpu as pltpu
from jax.experimental.pallas import tpu_sc as plsc

_SIZE = 32000
_N = 4096
_SMOOTHING = 0.1
_CONF = 1.0 - _SMOOTHING
_S = _SMOOTHING / (_SIZE - 2)
_C0 = _CONF * math.log(_CONF) + (_SIZE - 2) * _S * math.log(_S)

_BR = 64

_NC = 2
_NS = 16
_NW = _NC * _NS
_BPW = _N // _NW
_L = 16


_H = _SIZE // 2


def _tc_body(xa_ref, xb_ref, tgt_ref, lse_ref, xt_ref, g_ref):
    xa = xa_ref[...]
    xb = xb_ref[...]
    m = jnp.maximum(jnp.max(xa, axis=1), jnp.max(xb, axis=1))
    se = (jnp.sum(jnp.exp(xa - m[:, None]), axis=1)
          + jnp.sum(jnp.exp(xb - m[:, None]), axis=1))
    lse_ref[0, 0, :] = m + jnp.log(se)
    g_ref[0, 0, :] = jnp.sum(xa, axis=1) + jnp.sum(xb, axis=1) - xa[:, 0]

    lane_iota = lax.broadcasted_iota(jnp.int32, (1, 128), 1)
    for r in range(_BR):
        t = tgt_ref[0, 0, r]
        ta = jnp.minimum(t, _H - 1)
        tb = jnp.clip(t - _H, 0, _H - 1)
        cha = pl.multiple_of((ta // 128) * 128, 128)
        chb = pl.multiple_of((tb // 128) * 128, 128)
        va = xa_ref[pl.ds(r, 1), pl.ds(cha, 128)]
        vb = xb_ref[pl.ds(r, 1), pl.ds(chb, 128)]
        sa = jnp.sum(jnp.where(lane_iota == ta - cha, va, 0.0))
        sb = jnp.sum(jnp.where(lane_iota == tb - chb, vb, 0.0))
        xt_ref[0, 0, r] = jnp.where(t < _H, sa, sb)


def _sc_combine_body(lse_hbm, xt_hbm, g_hbm, tgt_hbm, out_hbm,
                     lse_v, xt_v, g_v, tgt_v, acc_v):
    wid = lax.axis_index("s") * _NC + lax.axis_index("c")
    base = wid * _BPW
    pltpu.sync_copy(lse_hbm.at[pl.ds(base, _BPW)], lse_v)
    pltpu.sync_copy(xt_hbm.at[pl.ds(base, _BPW)], xt_v)
    pltpu.sync_copy(g_hbm.at[pl.ds(base, _BPW)], g_v)
    pltpu.sync_copy(tgt_hbm.at[pl.ds(base, _BPW)], tgt_v)
    acc = jnp.zeros((_L,), jnp.float32)
    for c in range(_BPW // _L):
        sl = pl.ds(c * _L, _L)
        lse = lse_v[sl]
        xt = xt_v[sl]
        g = g_v[sl]
        tgt = tgt_v[sl]
        kl = _C0 + lse - _CONF * xt - _S * (g - xt)
        acc = acc + jnp.where(tgt != 0, kl, 0.0)
    acc_v[...] = acc
    pltpu.sync_copy(acc_v, out_hbm.at[wid])


_sc_combine = functools.partial(
    pl.kernel,
    mesh=plsc.VectorSubcoreMesh(core_axis_name="c", subcore_axis_name="s"),
    out_type=jax.ShapeDtypeStruct((_NW, _L), jnp.float32),
    scratch_types=[
        pltpu.VMEM((_BPW,), jnp.float32),
        pltpu.VMEM((_BPW,), jnp.float32),
        pltpu.VMEM((_BPW,), jnp.float32),
        pltpu.VMEM((_BPW,), jnp.int32),
        pltpu.VMEM((_L,), jnp.float32),
    ],
)(_sc_combine_body)


def _tc_final_body(p_ref, out_ref):
    out_ref[0] = jnp.sum(p_ref[...])


@jax.jit
def kernel(x, target):
    n, size = x.shape
    grid = n // _BR
    shp3 = jax.ShapeDtypeStruct((grid, 1, _BR), jnp.float32)
    blk3 = pl.BlockSpec((1, 1, _BR), lambda i: (i, 0, 0))
    lse3, xt3, g3 = pl.pallas_call(
        _tc_body,
        grid=(grid,),
        in_specs=[
            pl.BlockSpec((_BR, _H), lambda i: (i, 0)),
            pl.BlockSpec((_BR, _H), lambda i: (i, 1)),
            pl.BlockSpec((1, 1, _BR), lambda i: (i, 0, 0),
                         memory_space=pltpu.SMEM),
        ],
        out_specs=[blk3,
                   pl.BlockSpec((1, 1, _BR), lambda i: (i, 0, 0),
                                memory_space=pltpu.SMEM),
                   blk3],
        out_shape=[shp3, shp3, shp3],
    )(x, x, target.reshape(grid, 1, _BR))
    partials = _sc_combine(lse3.reshape(n), xt3.reshape(n), g3.reshape(n),
                           target)
    out = pl.pallas_call(
        _tc_final_body,
        out_specs=pl.BlockSpec(memory_space=pltpu.SMEM),
        out_shape=jax.ShapeDtypeStruct((1,), jnp.float32),
    )(partials)
    return out[0]


# speedup vs baseline: 1.0443x; 1.0443x over previous
import functools
import math

import jax
import jax.numpy as jnp
from jax import lax
from jax.experimental import pallas as pl
from jax.experimental.pallas import tpu as pltpu
from jax.experimental.pallas import tpu_sc as plsc

_SIZE = 32000
_N = 4096
_SMOOTHING = 0.1
_CONF = 1.0 - _SMOOTHING
_S = _SMOOTHING / (_SIZE - 2)
_C0 = _CONF * math.log(_CONF) + (_SIZE - 2) * _S * math.log(_S)

_BR = 64

_NC = 2
_NS = 16
_NW = _NC * _NS
_BPW = _N // _NW
_L = 16


def _tc_body(x_ref, tgt_ref, lse_ref, g_ref, xts_ref, xtacc_ref):
    i = pl.program_id(0)

    @pl.when(i == 0)
    def _init():
        xtacc_ref[...] = jnp.zeros((1, 128), jnp.float32)

    xb = x_ref[...]
    m = jnp.max(xb, axis=1)
    se = jnp.sum(jnp.exp(xb - m[:, None]), axis=1)
    lse_ref[0, 0, :] = m + jnp.log(se)
    g_ref[0, 0, :] = jnp.sum(xb, axis=1) - xb[:, 0]

    lane_iota = lax.broadcasted_iota(jnp.int32, (1, 128), 1)
    accs = [jnp.zeros((1, 128), jnp.float32) for _ in range(8)]
    for r in range(_BR):
        t = tgt_ref[0, 0, r]
        ch = pl.multiple_of((t // 128) * 128, 128)
        v = x_ref[pl.ds(r, 1), pl.ds(ch, 128)]
        cond = jnp.logical_and(lane_iota == t - ch, t != 0)
        accs[r % 8] = accs[r % 8] + jnp.where(cond, v, 0.0)
    total = accs[0]
    for k in range(1, 8):
        total = total + accs[k]
    xtacc_ref[...] += total

    @pl.when(i == pl.num_programs(0) - 1)
    def _fin():
        xts_ref[...] = xtacc_ref[...]


def _sc_combine_body(lse_hbm, g_hbm, tgt_hbm, out_hbm,
                     lse_v, g_v, tgt_v, acc_v):
    wid = lax.axis_index("s") * _NC + lax.axis_index("c")
    base = wid * _BPW
    pltpu.sync_copy(lse_hbm.at[pl.ds(base, _BPW)], lse_v)
    pltpu.sync_copy(g_hbm.at[pl.ds(base, _BPW)], g_v)
    pltpu.sync_copy(tgt_hbm.at[pl.ds(base, _BPW)], tgt_v)
    acc = jnp.zeros((_L,), jnp.float32)
    for c in range(_BPW // _L):
        sl = pl.ds(c * _L, _L)
        kl = _C0 + lse_v[sl] - _S * g_v[sl]
        acc = acc + jnp.where(tgt_v[sl] != 0, kl, 0.0)
    acc_v[...] = acc
    pltpu.sync_copy(acc_v, out_hbm.at[wid])


_sc_combine = functools.partial(
    pl.kernel,
    mesh=plsc.VectorSubcoreMesh(core_axis_name="c", subcore_axis_name="s"),
    out_type=jax.ShapeDtypeStruct((_NW, _L), jnp.float32),
    scratch_types=[
        pltpu.VMEM((_BPW,), jnp.float32),
        pltpu.VMEM((_BPW,), jnp.float32),
        pltpu.VMEM((_BPW,), jnp.int32),
        pltpu.VMEM((_L,), jnp.float32),
    ],
)(_sc_combine_body)


def _tc_final_body(p_ref, xts_ref, out_ref):
    out_ref[0] = jnp.sum(p_ref[...]) + (_S - _CONF) * jnp.sum(xts_ref[...])


@jax.jit
def kernel(x, target):
    n, size = x.shape
    grid = n // _BR
    shp3 = jax.ShapeDtypeStruct((grid, 1, _BR), jnp.float32)
    blk3 = pl.BlockSpec((1, 1, _BR), lambda i: (i, 0, 0))
    lse3, g3, xts = pl.pallas_call(
        _tc_body,
        grid=(grid,),
        in_specs=[
            pl.BlockSpec((_BR, size), lambda i: (i, 0)),
            pl.BlockSpec((1, 1, _BR), lambda i: (i, 0, 0),
                         memory_space=pltpu.SMEM),
        ],
        out_specs=[blk3, blk3,
                   pl.BlockSpec((1, 128), lambda i: (0, 0))],
        out_shape=[shp3, shp3,
                   jax.ShapeDtypeStruct((1, 128), jnp.float32)],
        scratch_shapes=[pltpu.VMEM((1, 128), jnp.float32)],
    )(x, target.reshape(grid, 1, _BR))
    partials = _sc_combine(lse3.reshape(n), g3.reshape(n), target)
    out = pl.pallas_call(
        _tc_final_body,
        out_specs=pl.BlockSpec(memory_space=pltpu.SMEM),
        out_shape=jax.ShapeDtypeStruct((1,), jnp.float32),
    )(partials, xts)
    return out[0]


# speedup vs baseline: 1.1810x; 1.1309x over previous
import functools
import math

import jax
import jax.numpy as jnp
from jax import lax
from jax.experimental import pallas as pl
from jax.experimental.pallas import tpu as pltpu
from jax.experimental.pallas import tpu_sc as plsc

_SIZE = 32000
_N = 4096
_SMOOTHING = 0.1
_CONF = 1.0 - _SMOOTHING
_S = _SMOOTHING / (_SIZE - 2)
_C0 = _CONF * math.log(_CONF) + (_SIZE - 2) * _S * math.log(_S)

_BR = 128

_NC = 2
_NS = 16
_NW = _NC * _NS
_BPW = _N // _NW
_L = 16


def _tc_body(x_ref, tgt_ref, lse_ref, g_ref, xts_ref, xtacc_ref):
    i = pl.program_id(0)

    @pl.when(i == 0)
    def _init():
        xtacc_ref[...] = jnp.zeros((1, 128), jnp.float32)

    xb = x_ref[...]
    m = jnp.max(xb, axis=1)
    se = jnp.sum(jnp.exp(xb - m[:, None]), axis=1)
    lse_ref[0, 0, :] = m + jnp.log(se)
    g_ref[0, 0, :] = jnp.sum(xb, axis=1) - xb[:, 0]

    lane_iota = lax.broadcasted_iota(jnp.int32, (1, 128), 1)
    accs = [jnp.zeros((1, 128), jnp.float32) for _ in range(8)]
    for r in range(_BR):
        t = tgt_ref[0, 0, r]
        ch = pl.multiple_of((t // 128) * 128, 128)
        v = x_ref[pl.ds(r, 1), pl.ds(ch, 128)]
        cond = jnp.logical_and(lane_iota == t - ch, t != 0)
        accs[r % 8] = accs[r % 8] + jnp.where(cond, v, 0.0)
    total = accs[0]
    for k in range(1, 8):
        total = total + accs[k]
    xtacc_ref[...] += total

    @pl.when(i == pl.num_programs(0) - 1)
    def _fin():
        xts_ref[...] = xtacc_ref[...]


def _sc_combine_body(lse_hbm, g_hbm, tgt_hbm, out_hbm,
                     lse_v, g_v, tgt_v, acc_v):
    wid = lax.axis_index("s") * _NC + lax.axis_index("c")
    base = wid * _BPW
    pltpu.sync_copy(lse_hbm.at[pl.ds(base, _BPW)], lse_v)
    pltpu.sync_copy(g_hbm.at[pl.ds(base, _BPW)], g_v)
    pltpu.sync_copy(tgt_hbm.at[pl.ds(base, _BPW)], tgt_v)
    acc = jnp.zeros((_L,), jnp.float32)
    for c in range(_BPW // _L):
        sl = pl.ds(c * _L, _L)
        kl = _C0 + lse_v[sl] - _S * g_v[sl]
        acc = acc + jnp.where(tgt_v[sl] != 0, kl, 0.0)
    acc_v[...] = acc
    pltpu.sync_copy(acc_v, out_hbm.at[wid])


_sc_combine = functools.partial(
    pl.kernel,
    mesh=plsc.VectorSubcoreMesh(core_axis_name="c", subcore_axis_name="s"),
    out_type=jax.ShapeDtypeStruct((_NW, _L), jnp.float32),
    scratch_types=[
        pltpu.VMEM((_BPW,), jnp.float32),
        pltpu.VMEM((_BPW,), jnp.float32),
        pltpu.VMEM((_BPW,), jnp.int32),
        pltpu.VMEM((_L,), jnp.float32),
    ],
)(_sc_combine_body)


def _tc_final_body(p_ref, xts_ref, out_ref):
    out_ref[0] = jnp.sum(p_ref[...]) + (_S - _CONF) * jnp.sum(xts_ref[...])


@jax.jit
def kernel(x, target):
    n, size = x.shape
    grid = n // _BR
    shp3 = jax.ShapeDtypeStruct((grid, 1, _BR), jnp.float32)
    blk3 = pl.BlockSpec((1, 1, _BR), lambda i: (i, 0, 0))
    lse3, g3, xts = pl.pallas_call(
        _tc_body,
        grid=(grid,),
        in_specs=[
            pl.BlockSpec((_BR, size), lambda i: (i, 0)),
            pl.BlockSpec((1, 1, _BR), lambda i: (i, 0, 0),
                         memory_space=pltpu.SMEM),
        ],
        out_specs=[blk3, blk3,
                   pl.BlockSpec((1, 128), lambda i: (0, 0))],
        out_shape=[shp3, shp3,
                   jax.ShapeDtypeStruct((1, 128), jnp.float32)],
        scratch_shapes=[pltpu.VMEM((1, 128), jnp.float32)],
    )(x, target.reshape(grid, 1, _BR))
    partials = _sc_combine(lse3.reshape(n), g3.reshape(n), target)
    out = pl.pallas_call(
        _tc_final_body,
        out_specs=pl.BlockSpec(memory_space=pltpu.SMEM),
        out_shape=jax.ShapeDtypeStruct((1,), jnp.float32),
    )(partials, xts)
    return out[0]


# speedup vs baseline: 1.2076x; 1.0226x over previous
import functools
import math

import jax
import jax.numpy as jnp
from jax import lax
from jax.experimental import pallas as pl
from jax.experimental.pallas import tpu as pltpu
from jax.experimental.pallas import tpu_sc as plsc

_SIZE = 32000
_N = 4096
_SMOOTHING = 0.1
_CONF = 1.0 - _SMOOTHING
_S = _SMOOTHING / (_SIZE - 2)
_C0 = _CONF * math.log(_CONF) + (_SIZE - 2) * _S * math.log(_S)

_BR = 256

_NC = 2
_NS = 16
_NW = _NC * _NS
_BPW = _N // _NW
_L = 16


def _tc_body(x_ref, tgt_ref, lse_ref, g_ref, xts_ref, xtacc_ref):
    i = pl.program_id(0)

    @pl.when(i == 0)
    def _init():
        xtacc_ref[...] = jnp.zeros((1, 128), jnp.float32)

    xb = x_ref[...]
    m = jnp.max(xb, axis=1)
    se = jnp.sum(jnp.exp(xb - m[:, None]), axis=1)
    lse_ref[0, 0, :] = m + jnp.log(se)
    g_ref[0, 0, :] = jnp.sum(xb, axis=1) - xb[:, 0]

    lane_iota = lax.broadcasted_iota(jnp.int32, (1, 128), 1)
    accs = [jnp.zeros((1, 128), jnp.float32) for _ in range(8)]
    for r in range(_BR):
        t = tgt_ref[0, 0, r]
        ch = pl.multiple_of((t // 128) * 128, 128)
        v = x_ref[pl.ds(r, 1), pl.ds(ch, 128)]
        cond = jnp.logical_and(lane_iota == t - ch, t != 0)
        accs[r % 8] = accs[r % 8] + jnp.where(cond, v, 0.0)
    total = accs[0]
    for k in range(1, 8):
        total = total + accs[k]
    xtacc_ref[...] += total

    @pl.when(i == pl.num_programs(0) - 1)
    def _fin():
        xts_ref[...] = xtacc_ref[...]


def _sc_combine_body(lse_hbm, g_hbm, tgt_hbm, out_hbm,
                     lse_v, g_v, tgt_v, acc_v):
    wid = lax.axis_index("s") * _NC + lax.axis_index("c")
    base = wid * _BPW
    pltpu.sync_copy(lse_hbm.at[pl.ds(base, _BPW)], lse_v)
    pltpu.sync_copy(g_hbm.at[pl.ds(base, _BPW)], g_v)
    pltpu.sync_copy(tgt_hbm.at[pl.ds(base, _BPW)], tgt_v)
    acc = jnp.zeros((_L,), jnp.float32)
    for c in range(_BPW // _L):
        sl = pl.ds(c * _L, _L)
        kl = _C0 + lse_v[sl] - _S * g_v[sl]
        acc = acc + jnp.where(tgt_v[sl] != 0, kl, 0.0)
    acc_v[...] = acc
    pltpu.sync_copy(acc_v, out_hbm.at[wid])


_sc_combine = functools.partial(
    pl.kernel,
    mesh=plsc.VectorSubcoreMesh(core_axis_name="c", subcore_axis_name="s"),
    out_type=jax.ShapeDtypeStruct((_NW, _L), jnp.float32),
    scratch_types=[
        pltpu.VMEM((_BPW,), jnp.float32),
        pltpu.VMEM((_BPW,), jnp.float32),
        pltpu.VMEM((_BPW,), jnp.int32),
        pltpu.VMEM((_L,), jnp.float32),
    ],
)(_sc_combine_body)


def _tc_final_body(p_ref, xts_ref, out_ref):
    out_ref[0] = jnp.sum(p_ref[...]) + (_S - _CONF) * jnp.sum(xts_ref[...])


@jax.jit
def kernel(x, target):
    n, size = x.shape
    grid = n // _BR
    shp3 = jax.ShapeDtypeStruct((grid, 1, _BR), jnp.float32)
    blk3 = pl.BlockSpec((1, 1, _BR), lambda i: (i, 0, 0))
    lse3, g3, xts = pl.pallas_call(
        _tc_body,
        grid=(grid,),
        in_specs=[
            pl.BlockSpec((_BR, size), lambda i: (i, 0)),
            pl.BlockSpec((1, 1, _BR), lambda i: (i, 0, 0),
                         memory_space=pltpu.SMEM),
        ],
        out_specs=[blk3, blk3,
                   pl.BlockSpec((1, 128), lambda i: (0, 0))],
        out_shape=[shp3, shp3,
                   jax.ShapeDtypeStruct((1, 128), jnp.float32)],
        scratch_shapes=[pltpu.VMEM((1, 128), jnp.float32)],
        compiler_params=pltpu.CompilerParams(
            vmem_limit_bytes=100 * 1024 * 1024),
    )(x, target.reshape(grid, 1, _BR))
    partials = _sc_combine(lse3.reshape(n), g3.reshape(n), target)
    out = pl.pallas_call(
        _tc_final_body,
        out_specs=pl.BlockSpec(memory_space=pltpu.SMEM),
        out_shape=jax.ShapeDtypeStruct((1,), jnp.float32),
    )(partials, xts)
    return out[0]
